# baseline (speedup 1.0000x reference)
import functools

import jax
import jax.numpy as jnp
from jax import lax
from jax.experimental import pallas as pl
from jax.experimental.pallas import tpu as pltpu
from jax.experimental.pallas import tpu_sc as plsc

_MARGIN = 0.6
_SCALE = 30.0
_NEG = -1e30

_B = 4096
_C = 11014
_CSC = 1024
_CH = 1024
_NTC = 10
_FULL = _CSC // 16

_NC, _NS = 2, 16
_NW = _NC * _NS
_BPW = _B // _NW


def _tc_body(lab_ref, *refs):
    x_refs = refs[:_NTC]
    m_ref, s_ref, t_ref = refs[_NTC:]
    lab = lab_ref[...]

    def stream(x, col0, mask_oob):
        cols = col0 + lax.broadcasted_iota(jnp.int32, x.shape, 1)
        is_t = cols == lab
        if mask_oob:
            x = jnp.where(cols < _C, x, _NEG)
        m = jnp.max(x, axis=1, keepdims=True)
        s = jnp.sum(jnp.exp(x * _SCALE - m * _SCALE), axis=1, keepdims=True)
        t = jnp.sum(jnp.where(is_t, x, 0.0), axis=1, keepdims=True)
        return m, s, t

    ms, ss, ts = zip(*[
        stream(x_refs[i][...], (i + 1) * _CH, i == _NTC - 1)
        for i in range(_NTC)
    ])
    m = functools.reduce(jnp.maximum, ms)
    s = sum(si * jnp.exp((mi - m) * _SCALE) for si, mi in zip(ss, ms))
    m_ref[...] = m * _SCALE
    s_ref[...] = s
    t_ref[...] = sum(ts) * _SCALE


def _make_tc_call(rows, interpret=False):
    rb = _B // rows
    return pl.pallas_call(
        _tc_body,
        grid=(rb,),
        in_specs=[pl.BlockSpec((rows, 1), lambda r: (r, 0))] + [
            pl.BlockSpec((rows, _CH), lambda r, i=i: (r, i + 1))
            for i in range(_NTC)
        ],
        out_specs=[
            pl.BlockSpec((rows, 1), lambda r: (r, 0)),
            pl.BlockSpec((rows, 1), lambda r: (r, 0)),
            pl.BlockSpec((rows, 1), lambda r: (r, 0)),
        ],
        out_shape=[
            jax.ShapeDtypeStruct((_B, 1), jnp.float32),
            jax.ShapeDtypeStruct((_B, 1), jnp.float32),
            jax.ShapeDtypeStruct((_B, 1), jnp.float32),
        ],
        compiler_params=pltpu.CompilerParams(
            dimension_semantics=("arbitrary",),
        ),
        interpret=interpret,
    )


def _sc_body(x_hbm, lab_hbm, m_hbm, s_hbm, t_hbm,
             labv, buf0, buf1, mg, sg, tb, sem0, sem1):
    wid = lax.axis_index("s") * _NC + lax.axis_index("c")
    base = wid * _BPW
    pltpu.sync_copy(lab_hbm.at[pl.ds(base, _BPW)], labv)
    lanes = lax.iota(jnp.int32, 16)
    bufs = (buf0, buf1)
    sems = (sem0, sem1)
    ngroups = _BPW // 16
    neg = jnp.full((16,), _NEG, jnp.float32)
    zero = jnp.zeros((16,), jnp.float32)

    def src(g):
        return x_hbm.at[pl.ds(base + g * 16, 16), pl.ds(0, _CSC)]

    pltpu.async_copy(src(0), buf0, sem0)
    pltpu.async_copy(src(1), buf1, sem1)

    @pl.loop(0, ngroups, step=2)
    def _groups(g):
        for b in range(2):
            gg = g + b
            buf = bufs[b]
            pltpu.make_async_copy(src(gg), buf, sems[b]).wait()
            g16 = pl.multiple_of(gg * 16, 16)
            lab_g = labv[pl.ds(g16, 16)]
            tg = zero
            for r in range(16):
                @pl.loop(0, _FULL, step=2, unroll=4, init_carry=(neg, neg))
                def mx(k, c, r=r, buf=buf):
                    m0, m1 = c
                    k16 = pl.multiple_of(k * 16, 16)
                    m0 = jnp.maximum(m0, buf[r, pl.ds(k16, 16)])
                    m1 = jnp.maximum(m1, buf[r, pl.ds(k16 + 16, 16)])
                    return (m0, m1)

                mm = jnp.maximum(mx[0], mx[1]) * _SCALE

                @pl.loop(0, _FULL, step=2, unroll=4, init_carry=(zero, zero))
                def sm(k, c, r=r, buf=buf, mm=mm):
                    s0, s1 = c
                    k16 = pl.multiple_of(k * 16, 16)
                    s0 = s0 + jnp.exp(buf[r, pl.ds(k16, 16)] * _SCALE - mm)
                    s1 = s1 + jnp.exp(buf[r, pl.ds(k16 + 16, 16)] * _SCALE - mm)
                    return (s0, s1)

                mg[r, :] = mm
                sg[r, :] = sm[0] + sm[1]

                labr = lab_g[r]
                cidx = jnp.clip(labr, 0, _CSC - 1)
                st = pl.multiple_of((cidx // 16) * 16, 16)
                chunk = buf[r, pl.ds(st, 16)]
                idxv = jnp.zeros((16,), jnp.int32) + (cidx - st)
                tv16 = lax.gather(
                    chunk, idxv[:, None],
                    lax.GatherDimensionNumbers(
                        offset_dims=(), collapsed_slice_dims=(0,),
                        start_index_map=(0,)),
                    (1,), mode=lax.GatherScatterMode.PROMISE_IN_BOUNDS)
                tg = jnp.where(lanes == r, tv16 * _SCALE, tg)

            tb[...] = jnp.where(lab_g < _CSC, tg, 0.0)

            @pl.when(gg + 2 < ngroups)
            def _():
                pltpu.async_copy(src(gg + 2), buf, sems[b])

            row_out = pl.ds(base + g16, 16)
            pltpu.sync_copy(mg, m_hbm.at[row_out])
            pltpu.sync_copy(sg, s_hbm.at[row_out])
            pltpu.sync_copy(tb, t_hbm.at[row_out])


def _make_sc_call():
    mesh = plsc.VectorSubcoreMesh(core_axis_name="c", subcore_axis_name="s")
    return functools.partial(
        pl.kernel,
        mesh=mesh,
        out_type=[
            jax.ShapeDtypeStruct((_B, 16), jnp.float32),
            jax.ShapeDtypeStruct((_B, 16), jnp.float32),
            jax.ShapeDtypeStruct((_B,), jnp.float32),
        ],
        scratch_types=[
            pltpu.VMEM((_BPW,), jnp.int32),
            pltpu.VMEM((16, _CSC), jnp.float32),
            pltpu.VMEM((16, _CSC), jnp.float32),
            pltpu.VMEM((16, 16), jnp.float32),
            pltpu.VMEM((16, 16), jnp.float32),
            pltpu.VMEM((16,), jnp.float32),
            pltpu.SemaphoreType.DMA,
            pltpu.SemaphoreType.DMA,
        ],
    )(_sc_body)


@jax.jit
def kernel(logits, labels):
    lab32 = labels.astype(jnp.int32)
    lab2d = lab32.reshape(_B, 1)

    m_scl, s_scl, t_sc = _make_sc_call()(logits, lab32)
    m_tc, s_tc, t_tc = _make_tc_call(rows=256)(lab2d, *([logits] * _NTC))

    m_tc = m_tc[:, 0]
    s_tc = s_tc[:, 0]
    t = t_tc[:, 0] + t_sc

    m_sc = jnp.max(m_scl, axis=1)
    s_sc = jnp.sum(s_scl * jnp.exp(m_scl - m_sc[:, None]), axis=1)

    m = jnp.maximum(m_tc, m_sc)
    s = s_tc * jnp.exp(m_tc - m) + s_sc * jnp.exp(m_sc - m)

    tm = t - _SCALE * _MARGIN
    s_adj = s - jnp.exp(t - m) + jnp.exp(tm - m)
    return jnp.mean(m + jnp.log(s_adj) - tm)


# speedup vs baseline: 1.0108x; 1.0108x over previous
import functools

import jax
import jax.numpy as jnp
from jax import lax
from jax.experimental import pallas as pl
from jax.experimental.pallas import tpu as pltpu
from jax.experimental.pallas import tpu_sc as plsc

_MARGIN = 0.6
_SCALE = 30.0
_NEG = -1e30

_B = 4096
_C = 11014
_CSC = 2048
_CH = 2048
_NTC = 5
_FULL = _CSC // 16

_NC, _NS = 2, 16
_NW = _NC * _NS
_BPW = _B // _NW


def _tc_body(lab_ref, *refs):
    x_refs = refs[:_NTC]
    m_ref, s_ref, t_ref = refs[_NTC:]
    lab = lab_ref[...]

    def stream(x, col0, mask_oob):
        cols = col0 + lax.broadcasted_iota(jnp.int32, x.shape, 1)
        is_t = cols == lab
        if mask_oob:
            x = jnp.where(cols < _C, x, _NEG)
        m = jnp.max(x, axis=1, keepdims=True)
        s = jnp.sum(jnp.exp(x * _SCALE - m * _SCALE), axis=1, keepdims=True)
        t = jnp.sum(jnp.where(is_t, x, 0.0), axis=1, keepdims=True)
        return m, s, t

    ms, ss, ts = zip(*[
        stream(x_refs[i][...], (i + 1) * _CH, i == _NTC - 1)
        for i in range(_NTC)
    ])
    m = functools.reduce(jnp.maximum, ms)
    s = sum(si * jnp.exp((mi - m) * _SCALE) for si, mi in zip(ss, ms))
    m_ref[...] = m * _SCALE
    s_ref[...] = s
    t_ref[...] = sum(ts) * _SCALE


def _make_tc_call(rows, interpret=False):
    rb = _B // rows
    return pl.pallas_call(
        _tc_body,
        grid=(rb,),
        in_specs=[pl.BlockSpec((rows, 1), lambda r: (r, 0))] + [
            pl.BlockSpec((rows, _CH), lambda r, i=i: (r, i + 1))
            for i in range(_NTC)
        ],
        out_specs=[
            pl.BlockSpec((rows, 1), lambda r: (r, 0)),
            pl.BlockSpec((rows, 1), lambda r: (r, 0)),
            pl.BlockSpec((rows, 1), lambda r: (r, 0)),
        ],
        out_shape=[
            jax.ShapeDtypeStruct((_B, 1), jnp.float32),
            jax.ShapeDtypeStruct((_B, 1), jnp.float32),
            jax.ShapeDtypeStruct((_B, 1), jnp.float32),
        ],
        compiler_params=pltpu.CompilerParams(
            dimension_semantics=("arbitrary",),
        ),
        interpret=interpret,
    )


def _sc_body(x_hbm, lab_hbm, m_hbm, s_hbm, t_hbm,
             labv, buf0, buf1, mg, sg, tb, sem0, sem1):
    wid = lax.axis_index("s") * _NC + lax.axis_index("c")
    base = wid * _BPW
    pltpu.sync_copy(lab_hbm.at[pl.ds(base, _BPW)], labv)
    lanes = lax.iota(jnp.int32, 16)
    bufs = (buf0, buf1)
    sems = (sem0, sem1)
    ngroups = _BPW // 16
    neg = jnp.full((16,), _NEG, jnp.float32)
    zero = jnp.zeros((16,), jnp.float32)

    def src(g):
        return x_hbm.at[pl.ds(base + g * 16, 16), pl.ds(0, _CSC)]

    pltpu.async_copy(src(0), buf0, sem0)
    pltpu.async_copy(src(1), buf1, sem1)

    @pl.loop(0, ngroups, step=2)
    def _groups(g):
        for b in range(2):
            gg = g + b
            buf = bufs[b]
            pltpu.make_async_copy(src(gg), buf, sems[b]).wait()
            g16 = pl.multiple_of(gg * 16, 16)
            lab_g = labv[pl.ds(g16, 16)]
            tg = zero
            for r in range(16):
                @pl.loop(0, _FULL, step=2, unroll=4, init_carry=(neg, neg))
                def mx(k, c, r=r, buf=buf):
                    m0, m1 = c
                    k16 = pl.multiple_of(k * 16, 16)
                    m0 = jnp.maximum(m0, buf[r, pl.ds(k16, 16)])
                    m1 = jnp.maximum(m1, buf[r, pl.ds(k16 + 16, 16)])
                    return (m0, m1)

                mm = jnp.maximum(mx[0], mx[1]) * _SCALE

                @pl.loop(0, _FULL, step=2, unroll=4, init_carry=(zero, zero))
                def sm(k, c, r=r, buf=buf, mm=mm):
                    s0, s1 = c
                    k16 = pl.multiple_of(k * 16, 16)
                    s0 = s0 + jnp.exp(buf[r, pl.ds(k16, 16)] * _SCALE - mm)
                    s1 = s1 + jnp.exp(buf[r, pl.ds(k16 + 16, 16)] * _SCALE - mm)
                    return (s0, s1)

                mg[r, :] = mm
                sg[r, :] = sm[0] + sm[1]

                labr = lab_g[r]
                cidx = jnp.clip(labr, 0, _CSC - 1)
                st = pl.multiple_of((cidx // 16) * 16, 16)
                chunk = buf[r, pl.ds(st, 16)]
                idxv = jnp.zeros((16,), jnp.int32) + (cidx - st)
                tv16 = lax.gather(
                    chunk, idxv[:, None],
                    lax.GatherDimensionNumbers(
                        offset_dims=(), collapsed_slice_dims=(0,),
                        start_index_map=(0,)),
                    (1,), mode=lax.GatherScatterMode.PROMISE_IN_BOUNDS)
                tg = jnp.where(lanes == r, tv16 * _SCALE, tg)

            tb[...] = jnp.where(lab_g < _CSC, tg, 0.0)

            @pl.when(gg + 2 < ngroups)
            def _():
                pltpu.async_copy(src(gg + 2), buf, sems[b])

            row_out = pl.ds(base + g16, 16)
            pltpu.sync_copy(mg, m_hbm.at[row_out])
            pltpu.sync_copy(sg, s_hbm.at[row_out])
            pltpu.sync_copy(tb, t_hbm.at[row_out])


def _make_sc_call():
    mesh = plsc.VectorSubcoreMesh(core_axis_name="c", subcore_axis_name="s")
    return functools.partial(
        pl.kernel,
        mesh=mesh,
        out_type=[
            jax.ShapeDtypeStruct((_B, 16), jnp.float32),
            jax.ShapeDtypeStruct((_B, 16), jnp.float32),
            jax.ShapeDtypeStruct((_B,), jnp.float32),
        ],
        scratch_types=[
            pltpu.VMEM((_BPW,), jnp.int32),
            pltpu.VMEM((16, _CSC), jnp.float32),
            pltpu.VMEM((16, _CSC), jnp.float32),
            pltpu.VMEM((16, 16), jnp.float32),
            pltpu.VMEM((16, 16), jnp.float32),
            pltpu.VMEM((16,), jnp.float32),
            pltpu.SemaphoreType.DMA,
            pltpu.SemaphoreType.DMA,
        ],
    )(_sc_body)


@jax.jit
def kernel(logits, labels):
    lab32 = labels.astype(jnp.int32)
    lab2d = lab32.reshape(_B, 1)

    m_scl, s_scl, t_sc = _make_sc_call()(logits, lab32)
    m_tc, s_tc, t_tc = _make_tc_call(rows=256)(lab2d, *([logits] * _NTC))

    m_tc = m_tc[:, 0]
    s_tc = s_tc[:, 0]
    t = t_tc[:, 0] + t_sc

    m_sc = jnp.max(m_scl, axis=1)
    s_sc = jnp.sum(s_scl * jnp.exp(m_scl - m_sc[:, None]), axis=1)

    m = jnp.maximum(m_tc, m_sc)
    s = s_tc * jnp.exp(m_tc - m) + s_sc * jnp.exp(m_sc - m)

    tm = t - _SCALE * _MARGIN
    s_adj = s - jnp.exp(t - m) + jnp.exp(tm - m)
    return jnp.mean(m + jnp.log(s_adj) - tm)


# speedup vs baseline: 1.0129x; 1.0021x over previous
import functools

import jax
import jax.numpy as jnp
from jax import lax
from jax.experimental import pallas as pl
from jax.experimental.pallas import tpu as pltpu
from jax.experimental.pallas import tpu_sc as plsc

_MARGIN = 0.6
_SCALE = 30.0
_NEG = -1e30

_B = 4096
_C = 11014
_CSC = 2048
_CH = 2048
_NTC = 5
_FULL = _CSC // 16

_NC, _NS = 2, 16
_NW = _NC * _NS
_BPW = _B // _NW


def _tc_body(lab_ref, *refs):
    x_refs = refs[:_NTC]
    m_ref, s_ref, t_ref = refs[_NTC:]
    lab = lab_ref[...]

    def stream(x, col0, mask_oob):
        cols = col0 + lax.broadcasted_iota(jnp.int32, x.shape, 1)
        is_t = cols == lab
        if mask_oob:
            x = jnp.where(cols < _C, x, _NEG)
        m = jnp.max(x, axis=1, keepdims=True)
        s = jnp.sum(jnp.exp(x * _SCALE - m * _SCALE), axis=1, keepdims=True)
        t = jnp.sum(jnp.where(is_t, x, 0.0), axis=1, keepdims=True)
        return m, s, t

    ms, ss, ts = zip(*[
        stream(x_refs[i][...], (i + 1) * _CH, i == _NTC - 1)
        for i in range(_NTC)
    ])
    m = functools.reduce(jnp.maximum, ms)
    s = sum(si * jnp.exp((mi - m) * _SCALE) for si, mi in zip(ss, ms))
    m_ref[...] = m * _SCALE
    s_ref[...] = s
    t_ref[...] = sum(ts) * _SCALE


def _make_tc_call(rows, interpret=False):
    rb = _B // rows
    return pl.pallas_call(
        _tc_body,
        grid=(rb,),
        in_specs=[pl.BlockSpec((rows, 1), lambda r: (r, 0))] + [
            pl.BlockSpec((rows, _CH), lambda r, i=i: (r, i + 1))
            for i in range(_NTC)
        ],
        out_specs=[
            pl.BlockSpec((rows, 1), lambda r: (r, 0)),
            pl.BlockSpec((rows, 1), lambda r: (r, 0)),
            pl.BlockSpec((rows, 1), lambda r: (r, 0)),
        ],
        out_shape=[
            jax.ShapeDtypeStruct((_B, 1), jnp.float32),
            jax.ShapeDtypeStruct((_B, 1), jnp.float32),
            jax.ShapeDtypeStruct((_B, 1), jnp.float32),
        ],
        compiler_params=pltpu.CompilerParams(
            dimension_semantics=("arbitrary",),
        ),
        interpret=interpret,
    )


def _sc_body(x_hbm, lab_hbm, m_hbm, s_hbm, t_hbm,
             labv, buf0, buf1, mg, sg, tb, sem0, sem1):
    wid = lax.axis_index("s") * _NC + lax.axis_index("c")
    base = wid * _BPW
    pltpu.sync_copy(lab_hbm.at[pl.ds(base, _BPW)], labv)
    lanes = lax.iota(jnp.int32, 16)
    bufs = (buf0, buf1)
    sems = (sem0, sem1)
    ngroups = _BPW // 16
    neg = jnp.full((16,), _NEG, jnp.float32)
    zero = jnp.zeros((16,), jnp.float32)

    def src(g):
        return x_hbm.at[pl.ds(base + g * 16, 16), pl.ds(0, _CSC)]

    pltpu.async_copy(src(0), buf0, sem0)
    pltpu.async_copy(src(1), buf1, sem1)

    @pl.loop(0, ngroups, step=2)
    def _groups(g):
        for b in range(2):
            gg = g + b
            buf = bufs[b]
            pltpu.make_async_copy(src(gg), buf, sems[b]).wait()
            g16 = pl.multiple_of(gg * 16, 16)
            lab_g = labv[pl.ds(g16, 16)]
            tg = zero
            for r in range(16):
                @pl.loop(0, _FULL, step=2, unroll=4, init_carry=(neg, neg))
                def mx(k, c, r=r, buf=buf):
                    m0, m1 = c
                    k16 = pl.multiple_of(k * 16, 16)
                    m0 = jnp.maximum(m0, buf[r, pl.ds(k16, 16)])
                    m1 = jnp.maximum(m1, buf[r, pl.ds(k16 + 16, 16)])
                    return (m0, m1)

                mm = jnp.maximum(mx[0], mx[1]) * _SCALE

                @pl.loop(0, _FULL, step=2, unroll=4, init_carry=(zero, zero))
                def sm(k, c, r=r, buf=buf, mm=mm):
                    s0, s1 = c
                    k16 = pl.multiple_of(k * 16, 16)
                    s0 = s0 + jnp.exp(buf[r, pl.ds(k16, 16)] * _SCALE - mm)
                    s1 = s1 + jnp.exp(buf[r, pl.ds(k16 + 16, 16)] * _SCALE - mm)
                    return (s0, s1)

                mg[r, :] = mm
                sg[r, :] = sm[0] + sm[1]

                labr = lab_g[r]
                cidx = jnp.clip(labr, 0, _CSC - 1)
                st = pl.multiple_of((cidx // 16) * 16, 16)
                chunk = buf[r, pl.ds(st, 16)]
                idxv = jnp.zeros((16,), jnp.int32) + (cidx - st)
                tv16 = lax.gather(
                    chunk, idxv[:, None],
                    lax.GatherDimensionNumbers(
                        offset_dims=(), collapsed_slice_dims=(0,),
                        start_index_map=(0,)),
                    (1,), mode=lax.GatherScatterMode.PROMISE_IN_BOUNDS)
                tg = jnp.where(lanes == r, tv16 * _SCALE, tg)

            tb[...] = jnp.where(lab_g < _CSC, tg, 0.0)

            @pl.when(gg + 2 < ngroups)
            def _():
                pltpu.async_copy(src(gg + 2), buf, sems[b])

            row_out = pl.ds(base + g16, 16)
            pltpu.sync_copy(mg, m_hbm.at[row_out])
            pltpu.sync_copy(sg, s_hbm.at[row_out])
            pltpu.sync_copy(tb, t_hbm.at[row_out])


def _make_sc_call():
    mesh = plsc.VectorSubcoreMesh(core_axis_name="c", subcore_axis_name="s")
    return functools.partial(
        pl.kernel,
        mesh=mesh,
        out_type=[
            jax.ShapeDtypeStruct((_B, 16), jnp.float32),
            jax.ShapeDtypeStruct((_B, 16), jnp.float32),
            jax.ShapeDtypeStruct((_B,), jnp.float32),
        ],
        scratch_types=[
            pltpu.VMEM((_BPW,), jnp.int32),
            pltpu.VMEM((16, _CSC), jnp.float32),
            pltpu.VMEM((16, _CSC), jnp.float32),
            pltpu.VMEM((16, 16), jnp.float32),
            pltpu.VMEM((16, 16), jnp.float32),
            pltpu.VMEM((16,), jnp.float32),
            pltpu.SemaphoreType.DMA,
            pltpu.SemaphoreType.DMA,
        ],
    )(_sc_body)


@jax.jit
def kernel(logits, labels):
    lab32 = labels.astype(jnp.int32)
    lab2d = lab32.reshape(_B, 1)

    m_scl, s_scl, t_sc = _make_sc_call()(logits, lab32)
    m_tc, s_tc, t_tc = _make_tc_call(rows=512)(lab2d, *([logits] * _NTC))

    m_tc = m_tc[:, 0]
    s_tc = s_tc[:, 0]
    t = t_tc[:, 0] + t_sc

    m_sc = jnp.max(m_scl, axis=1)
    s_sc = jnp.sum(s_scl * jnp.exp(m_scl - m_sc[:, None]), axis=1)

    m = jnp.maximum(m_tc, m_sc)
    s = s_tc * jnp.exp(m_tc - m) + s_sc * jnp.exp(m_sc - m)

    tm = t - _SCALE * _MARGIN
    s_adj = s - jnp.exp(t - m) + jnp.exp(tm - m)
    return jnp.mean(m + jnp.log(s_adj) - tm)
